# baseline (speedup 1.0000x reference)
import functools

import jax
import jax.numpy as jnp
from jax import lax
from jax.experimental import pallas as pl
from jax.experimental.pallas import tpu as pltpu
from jax.experimental.pallas import tpu_sc as plsc

_NC = 2
_NS = 16
_NW = _NC * _NS
_CH = 1024


def _silu(x):
    return x * jax.nn.sigmoid(x)


def _ln(x, g, b):
    m = x.mean(-1, keepdims=True)
    v = ((x - m) ** 2).mean(-1, keepdims=True)
    return (x - m) * jax.lax.rsqrt(v + 1e-5) * g + b


def _transform_body(nl_ref, nf_ref, w1a_ref, w1b_ref, b1_ref, nl2_ref, nf2_ref):
    nl2_ref[...] = jnp.dot(nl_ref[...], w1a_ref[...],
                           preferred_element_type=jnp.float32) + b1_ref[...]
    nf2_ref[...] = jnp.dot(nf_ref[...], w1b_ref[...],
                           preferred_element_type=jnp.float32)


def _transform_tables(node_latents, node_features, w1a, w1b, b1, block):
    n, h = node_latents.shape
    grid = (n // block,)
    full = lambda s: pl.BlockSpec(s, lambda i: (0,) * len(s))
    row = pl.BlockSpec((block, h), lambda i: (i, 0))
    return pl.pallas_call(
        _transform_body,
        grid=grid,
        in_specs=[row, row, full((h, h)), full((h, h)), full((1, h))],
        out_specs=[row, row],
        out_shape=[jax.ShapeDtypeStruct((n, h), jnp.float32)] * 2,
    )(node_latents, node_features, w1a, w1b, b1)


def _ee_t_body(xt_ref, w1t, b1c, w2t, b2c, gc, bec, eft_ref):
    xt = xt_ref[...]
    h1 = _silu(jnp.dot(w1t[...], xt, preferred_element_type=jnp.float32)
               + b1c[...])
    e1 = jnp.dot(w2t[...], h1, preferred_element_type=jnp.float32) + b2c[...]
    m = e1.mean(0, keepdims=True)
    v = ((e1 - m) ** 2).mean(0, keepdims=True)
    eft_ref[...] = (e1 - m) * jax.lax.rsqrt(v + 1e-5) * gc[...] + bec[...]


def _ee_transposed(xt, w1t, b1c, w2t, b2c, gc, bec, block):
    de, e = xt.shape
    h = w1t.shape[0]
    grid = (e // block,)
    full = lambda s: pl.BlockSpec(s, lambda i: (0,) * len(s))
    colx = pl.BlockSpec((de, block), lambda i: (0, i))
    colh = pl.BlockSpec((h, block), lambda i: (0, i))
    return pl.pallas_call(
        _ee_t_body,
        grid=grid,
        in_specs=[colx, full((h, de)), full((h, 1)), full((h, h)),
                  full((h, 1)), full((h, 1)), full((h, 1))],
        out_specs=colh,
        out_shape=jax.ShapeDtypeStruct((h, e), jnp.float32),
    )(xt, w1t, b1c, w2t, b2c, gc, bec)


def _sc_gather_add(nl2, nf2, s1d, r1d):
    h = nl2.shape[1]
    e_pad = s1d.shape[0]
    per_w = e_pad // _NW
    n_chunks = per_w // _CH
    cpk = _CH // 4
    mesh = plsc.VectorSubcoreMesh(core_axis_name="c", subcore_axis_name="s")

    @functools.partial(
        pl.kernel, mesh=mesh,
        compiler_params=pltpu.CompilerParams(use_tc_tiling_on_sc=False),
        out_type=[jax.ShapeDtypeStruct((e_pad // 4, 4 * h), jnp.float32),
                  jax.ShapeDtypeStruct((e_pad // 4, 4 * h), jnp.float32)],
        scratch_types=[
            pltpu.VMEM((_CH,), jnp.int32),
            pltpu.VMEM((_CH,), jnp.int32),
            pltpu.VMEM((_CH, h), jnp.float32),
            pltpu.VMEM((_CH, h), jnp.float32),
            pltpu.SemaphoreType.DMA,
            pltpu.SemaphoreType.DMA,
            pltpu.SemaphoreType.DMA,
        ])
    def k(nl2_hbm, nf2_hbm, s_hbm, r_hbm, outa_hbm, outb_hbm, sidx, ridx,
          bufa, bufb, sema, semb, semc):
        wid = lax.axis_index("s") * _NC + lax.axis_index("c")
        base_w = wid * per_w
        base_w4 = wid * (per_w // 4)

        @pl.loop(0, n_chunks)
        def _(ci):
            base = base_w + ci * _CH
            base4 = base_w4 + ci * cpk
            pltpu.sync_copy(s_hbm.at[pl.ds(base, _CH)], sidx)
            pltpu.sync_copy(r_hbm.at[pl.ds(base, _CH)], ridx)
            cpa = pltpu.async_copy(nl2_hbm.at[sidx], bufa, sema)
            cpb = pltpu.async_copy(nf2_hbm.at[ridx], bufb, semb)
            cpa.wait()
            cpb.wait()
            cps = []
            for buf, out in ((bufa, outa_hbm), (bufb, outb_hbm)):
                for j in range(4):
                    cps.append(pltpu.async_copy(
                        buf.at[pl.ds(j * cpk, cpk)],
                        out.at[pl.ds(base4, cpk), pl.ds(j * h, h)], semc))
            for cp in cps:
                cp.wait()

    return k(nl2, nf2, s1d, r1d)


def _sc_scatter(nef_p, r1d_perm, n, h):
    e_pad = nef_p.shape[0] * 4
    hh = h // 2
    per_t = e_pad // _NS
    n_chunks = per_t // _CH
    acc_rows = 102400
    stripe = acc_rows // _NS
    zb = 128
    last_flush = n - (_NS - 1) * stripe
    per_t_cnt = e_pad // _NC // _NS
    cnt_chunks = per_t_cnt // _CH
    mesh = plsc.VectorSubcoreMesh(core_axis_name="c", subcore_axis_name="s")

    @functools.partial(
        pl.kernel, mesh=mesh,
        compiler_params=pltpu.CompilerParams(use_tc_tiling_on_sc=False),
        out_type=[jax.ShapeDtypeStruct((n, h), jnp.float32),
                  jax.ShapeDtypeStruct((n, h), jnp.float32)],
        scratch_types=[
            pltpu.VMEM((_CH,), jnp.int32),
            pltpu.VMEM((_CH, hh), jnp.float32),
            pltpu.VMEM((zb, hh), jnp.float32),
            pltpu.VMEM((512, hh), jnp.float32),
            pltpu.VMEM_SHARED((acc_rows, hh), jnp.float32),
            pltpu.SemaphoreType.DMA,
        ])
    def k(nef_hbm, ridx_hbm, seg_hbm, cnt_hbm, idxb, valb, zerob, oneb, acc,
          sem):
        core = lax.axis_index("c")
        tid = lax.axis_index("s")
        colbase = core * hh

        @pl.loop(0, zb)
        def _(i):
            zerob[i, pl.ds(0, hh)] = jnp.zeros((hh,), jnp.float32)

        @pl.loop(0, 512)
        def _(i):
            oneb[i, pl.ds(0, hh)] = jnp.ones((hh,), jnp.float32)

        def zero_acc():
            for z in range(stripe // zb):
                pltpu.sync_copy(zerob, acc.at[pl.ds(tid * stripe + z * zb, zb)])

        def flush(dst_hbm):
            @pl.when(tid < _NS - 1)
            def _():
                pltpu.sync_copy(
                    acc.at[pl.ds(tid * stripe, stripe)],
                    dst_hbm.at[pl.ds(tid * stripe, stripe),
                               pl.ds(colbase, hh)])

            @pl.when(tid == _NS - 1)
            def _():
                pltpu.sync_copy(
                    acc.at[pl.ds((_NS - 1) * stripe, last_flush)],
                    dst_hbm.at[pl.ds((_NS - 1) * stripe, last_flush),
                               pl.ds(colbase, hh)])

        zero_acc()
        plsc.subcore_barrier()

        @pl.loop(0, n_chunks)
        def _(ci):
            base = tid * per_t + ci * _CH
            base4 = base // 4
            pltpu.sync_copy(ridx_hbm.at[pl.ds(base, _CH)], idxb)
            cps = []
            for j in range(4):
                cps.append(pltpu.async_copy(
                    nef_hbm.at[pl.ds(base4, _CH // 4),
                               pl.ds(j * h + colbase, hh)],
                    valb.at[pl.ds(j * (_CH // 4), _CH // 4)], sem))
            for cp in cps:
                cp.wait()
            pltpu.sync_copy(valb, acc.at[idxb], add=True)

        plsc.subcore_barrier()
        flush(seg_hbm)
        plsc.subcore_barrier()
        zero_acc()
        plsc.subcore_barrier()

        @pl.loop(0, cnt_chunks)
        def _(ci):
            base = core * (e_pad // _NC) + tid * per_t_cnt + ci * _CH
            pltpu.sync_copy(ridx_hbm.at[pl.ds(base, _CH)], idxb)
            pltpu.sync_copy(oneb, acc.at[idxb.at[pl.ds(0, 512)]], add=True)
            pltpu.sync_copy(oneb, acc.at[idxb.at[pl.ds(512, 512)]], add=True)

        plsc.subcore_barrier()
        flush(cnt_hbm)

    return k(nef_p, r1d_perm)


def _ln_packed(x, mavg, g, b):
    m = jnp.dot(x, mavg, preferred_element_type=jnp.float32)
    d = x - m
    v = jnp.dot(d * d, mavg, preferred_element_type=jnp.float32)
    return d * jax.lax.rsqrt(v + 1e-5) * g + b


def _edge_body(x_ref, ga_ref, gb_ref, w1bd, eeb1, w2bd, eeb2, eeg, eebe,
               w1cbd, pew2bd, peb2, peg, pebe, mavg_ref, nef_ref):
    x = x_ref[...]
    mavg = mavg_ref[...]
    h1 = _silu(jnp.dot(x, w1bd[...], preferred_element_type=jnp.float32)
               + eeb1[...])
    e1 = jnp.dot(h1, w2bd[...], preferred_element_type=jnp.float32) + eeb2[...]
    ef = _ln_packed(e1, mavg, eeg[...], eebe[...])
    pre = ga_ref[...] + gb_ref[...] + jnp.dot(ef, w1cbd[...],
                                               preferred_element_type=jnp.float32)
    h2 = _silu(pre)
    e2 = jnp.dot(h2, pew2bd[...], preferred_element_type=jnp.float32) + peb2[...]
    nef_ref[...] = _ln_packed(e2, mavg, peg[...], pebe[...])


def _edge_pass(x_p, ga_p, gb_p, w1bd, eeb1, w2bd, eeb2, eeg, eebe,
               w1cbd, pew2bd, peb2, peg, pebe, mavg, block):
    ep4, de4 = x_p.shape
    grid = (ep4 // block,)
    full = lambda s: pl.BlockSpec(s, lambda i: (0,) * len(s))
    rowx = pl.BlockSpec((block, de4), lambda i: (i, 0))
    rowp = pl.BlockSpec((block, 128), lambda i: (i, 0))
    return pl.pallas_call(
        _edge_body,
        grid=grid,
        in_specs=[rowx, rowp, rowp,
                  full((de4, 128)), full((1, 128)), full((128, 128)),
                  full((1, 128)), full((1, 128)), full((1, 128)),
                  full((128, 128)), full((128, 128)), full((1, 128)),
                  full((1, 128)), full((1, 128)), full((128, 128))],
        out_specs=rowp,
        out_shape=jax.ShapeDtypeStruct((ga_p.shape[0], 128), jnp.float32),
    )(x_p, ga_p, gb_p, w1bd, eeb1, w2bd, eeb2, eeg, eebe,
      w1cbd, pew2bd, peb2, peg, pebe, mavg)


def _node_body(nf_ref, seg_ref, cnt_ref, pnW1a, pnW1b, pnb1, pnW2, pnb2,
               png, pnbe, noW1, nob1, noW2, nob2, out_ref):
    cnt = cnt_ref[:, 0:1] + cnt_ref[:, 16:17]
    mean = seg_ref[...] / jnp.maximum(cnt, 1.0)
    nf = nf_ref[...]
    pre = (jnp.dot(nf, pnW1a[...], preferred_element_type=jnp.float32)
           + jnp.dot(mean, pnW1b[...], preferred_element_type=jnp.float32)
           + pnb1[...])
    hdd = _silu(pre)
    y = _ln(jnp.dot(hdd, pnW2[...], preferred_element_type=jnp.float32)
            + pnb2[...], png[...], pnbe[...])
    z = jax.nn.sigmoid(jnp.dot(y, noW1[...], preferred_element_type=jnp.float32)
                       + nob1[...])
    out_ref[...] = (jnp.dot(z, noW2[...], preferred_element_type=jnp.float32)
                    + nob2[...])


def _node_pass(node_features, seg_sum, counts, pn_W1, pn_b1, pn_W2, pn_b2,
               pn_g, pn_be, no_W1, no_b1, no_W2, no_b2, block):
    n, h = node_features.shape
    dout = no_W2.shape[1]
    grid = (n // block,)
    full = lambda s: pl.BlockSpec(s, lambda i: (0,) * len(s))
    rowh = pl.BlockSpec((block, h), lambda i: (i, 0))
    rowo = pl.BlockSpec((block, dout), lambda i: (i, 0))
    w1a, w1b = pn_W1[:h], pn_W1[h:]
    return pl.pallas_call(
        _node_body,
        grid=grid,
        in_specs=[rowh, rowh, rowh,
                  full((h, h)), full((h, h)), full((1, h)), full((h, h)),
                  full((1, h)), full((1, h)), full((1, h)),
                  full((h, h)), full((1, h)), full((h, dout)), full((1, dout))],
        out_specs=rowo,
        out_shape=jax.ShapeDtypeStruct((n, dout), jnp.float32),
    )(node_features, seg_sum, counts, w1a, w1b, pn_b1, pn_W2, pn_b2,
      pn_g, pn_be, no_W1, no_b1, no_W2, no_b2)


def kernel(edge_idx, edge_features, node_latents, node_features,
           ee_W1, ee_b1, ee_W2, ee_b2, ee_g, ee_be,
           pe_W1, pe_b1, pe_W2, pe_b2, pe_g, pe_be,
           pn_W1, pn_b1, pn_W2, pn_b2, pn_g, pn_be,
           no_W1, no_b1, no_W2, no_b2):
    e = edge_idx.shape[0]
    n, h = node_features.shape
    r2 = lambda v: v.reshape(1, -1)

    senders = edge_idx[:, 0]
    receivers = edge_idx[:, 1]
    e_pad = ((e + _CH * _NW - 1) // (_CH * _NW)) * (_CH * _NW)
    pad0 = jnp.zeros((e_pad - e,), jnp.int32)
    s1d = jnp.concatenate([senders, pad0])
    r1d_g = jnp.concatenate([receivers, pad0])
    r1d = jnp.concatenate([receivers, jnp.full((e_pad - e,), n, jnp.int32)])
    r1d_perm = r1d.reshape(-1, _CH // 4, 4).transpose(0, 2, 1).reshape(-1)

    from jax.scipy.linalg import block_diag
    bd4 = lambda w: block_diag(w, w, w, w)
    t4 = lambda v: jnp.tile(v, 4).reshape(1, -1)
    x_p = edge_features.reshape(e // 4, -1)
    mavg = bd4(jnp.full((h, h), 1.0 / h, jnp.float32))

    w1a, w1b, w1c = pe_W1[:h], pe_W1[h:2 * h], pe_W1[2 * h:]
    nl2, nf2 = _transform_tables(node_latents, node_features, w1a, w1b,
                                 r2(pe_b1), block=4000 if n % 4000 == 0 else n)

    perm = lambda a: a.reshape(-1, _CH // 4, 4).transpose(0, 2, 1).reshape(-1)
    ga_p, gb_p = _sc_gather_add(nl2, nf2, perm(s1d), perm(r1d_g))

    cvec = lambda v: v.reshape(-1, 1)
    eft = _ee_transposed(edge_features.T, ee_W1.T, cvec(ee_b1), ee_W2.T,
                         cvec(ee_b2), cvec(ee_g), cvec(ee_be),
                         block=6400 if e % 6400 == 0 else e)
    ef = eft.T

    nef_p = _edge_pass(x_p, ga_p, gb_p, bd4(ee_W1), t4(ee_b1), bd4(ee_W2),
                       t4(ee_b2), t4(ee_g), t4(ee_be), bd4(w1c),
                       bd4(pe_W2), t4(pe_b2), t4(pe_g), t4(pe_be),
                       mavg, block=2000 if (e // 4) % 2000 == 0 else e // 4)

    seg_sum, counts = _sc_scatter(nef_p, r1d_perm, n, h)

    out = _node_pass(node_features, seg_sum, counts, pn_W1, r2(pn_b1), pn_W2,
                     r2(pn_b2), r2(pn_g), r2(pn_be), no_W1, r2(no_b1), no_W2,
                     r2(no_b2), block=4000 if n % 4000 == 0 else n)
    return (ef, out)


# speedup vs baseline: 1.2498x; 1.2498x over previous
import functools

import jax
import jax.numpy as jnp
from jax import lax
from jax.experimental import pallas as pl
from jax.experimental.pallas import tpu as pltpu
from jax.experimental.pallas import tpu_sc as plsc

_NC = 2
_NS = 16
_NW = _NC * _NS
_CH = 1024


def _silu(x):
    return x * jax.nn.sigmoid(x)


def _ln(x, g, b):
    m = x.mean(-1, keepdims=True)
    v = ((x - m) ** 2).mean(-1, keepdims=True)
    return (x - m) * jax.lax.rsqrt(v + 1e-5) * g + b


def _transform_body(nl_ref, nf_ref, w1a_ref, w1b_ref, b1_ref, nl2_ref, nf2_ref):
    nl2_ref[...] = jnp.dot(nl_ref[...], w1a_ref[...],
                           preferred_element_type=jnp.float32) + b1_ref[...]
    nf2_ref[...] = jnp.dot(nf_ref[...], w1b_ref[...],
                           preferred_element_type=jnp.float32)


def _transform_tables(node_latents, node_features, w1a, w1b, b1, block):
    n, h = node_latents.shape
    grid = (n // block,)
    full = lambda s: pl.BlockSpec(s, lambda i: (0,) * len(s))
    row = pl.BlockSpec((block, h), lambda i: (i, 0))
    return pl.pallas_call(
        _transform_body,
        grid=grid,
        in_specs=[row, row, full((h, h)), full((h, h)), full((1, h))],
        out_specs=[row, row],
        out_shape=[jax.ShapeDtypeStruct((n, h), jnp.float32)] * 2,
    )(node_latents, node_features, w1a, w1b, b1)


def _ee_t_body(xt_ref, w1t, b1c, w2t, b2c, gc, bec, eft_ref):
    xt = xt_ref[...]
    h1 = _silu(jnp.dot(w1t[...], xt, preferred_element_type=jnp.float32)
               + b1c[...])
    e1 = jnp.dot(w2t[...], h1, preferred_element_type=jnp.float32) + b2c[...]
    m = e1.mean(0, keepdims=True)
    v = ((e1 - m) ** 2).mean(0, keepdims=True)
    eft_ref[...] = (e1 - m) * jax.lax.rsqrt(v + 1e-5) * gc[...] + bec[...]


def _ee_transposed(xt, w1t, b1c, w2t, b2c, gc, bec, block):
    de, e = xt.shape
    h = w1t.shape[0]
    grid = (e // block,)
    full = lambda s: pl.BlockSpec(s, lambda i: (0,) * len(s))
    colx = pl.BlockSpec((de, block), lambda i: (0, i))
    colh = pl.BlockSpec((h, block), lambda i: (0, i))
    return pl.pallas_call(
        _ee_t_body,
        grid=grid,
        in_specs=[colx, full((h, de)), full((h, 1)), full((h, h)),
                  full((h, 1)), full((h, 1)), full((h, 1))],
        out_specs=colh,
        out_shape=jax.ShapeDtypeStruct((h, e), jnp.float32),
    )(xt, w1t, b1c, w2t, b2c, gc, bec)


def _sc_gather_add(nl2, nf2, s1d, r1d):
    h = nl2.shape[1]
    e_pad = s1d.shape[0]
    per_w = e_pad // _NW
    n_chunks = per_w // _CH
    cpk = _CH // 4
    mesh = plsc.VectorSubcoreMesh(core_axis_name="c", subcore_axis_name="s")

    @functools.partial(
        pl.kernel, mesh=mesh,
        compiler_params=pltpu.CompilerParams(use_tc_tiling_on_sc=False),
        out_type=jax.ShapeDtypeStruct((e_pad // 4, 4 * h), jnp.float32),
        scratch_types=[
            pltpu.VMEM((_CH,), jnp.int32),
            pltpu.VMEM((_CH,), jnp.int32),
            pltpu.VMEM((_CH, h), jnp.float32),
            pltpu.VMEM((_CH, h), jnp.float32),
            pltpu.VMEM((cpk, 4 * h), jnp.float32),
            pltpu.SemaphoreType.DMA,
            pltpu.SemaphoreType.DMA,
        ])
    def k(nl2_hbm, nf2_hbm, s_hbm, r_hbm, out_hbm, sidx, ridx, bufa, bufb,
          packb, sema, semb):
        wid = lax.axis_index("s") * _NC + lax.axis_index("c")
        base_w = wid * per_w
        base_w4 = wid * (per_w // 4)

        @pl.loop(0, n_chunks)
        def _(ci):
            base = base_w + ci * _CH
            pltpu.sync_copy(s_hbm.at[pl.ds(base, _CH)], sidx)
            pltpu.sync_copy(r_hbm.at[pl.ds(base, _CH)], ridx)
            cpa = pltpu.async_copy(nl2_hbm.at[sidx], bufa, sema)
            cpb = pltpu.async_copy(nf2_hbm.at[ridx], bufb, semb)
            cpa.wait()
            cpb.wait()

            @pl.loop(0, cpk)
            def _(r):
                for j in range(4):
                    i = 4 * r + j
                    packb[r, pl.ds(j * h, 16)] = (bufa[i, pl.ds(0, 16)]
                                                  + bufb[i, pl.ds(0, 16)])
                    packb[r, pl.ds(j * h + 16, 16)] = (bufa[i, pl.ds(16, 16)]
                                                       + bufb[i, pl.ds(16, 16)])

            pltpu.sync_copy(packb, out_hbm.at[pl.ds(base_w4 + ci * cpk, cpk)])

    return k(nl2, nf2, s1d, r1d)


def _sc_scatter(nef_p, r1d_perm, n, h):
    e_pad = nef_p.shape[0] * 4
    hh = h // 2
    per_t = e_pad // _NS
    n_chunks = per_t // _CH
    acc_rows = 102400
    stripe = acc_rows // _NS
    zb = 128
    last_flush = n - (_NS - 1) * stripe
    per_t_cnt = e_pad // _NC // _NS
    cnt_chunks = per_t_cnt // _CH
    mesh = plsc.VectorSubcoreMesh(core_axis_name="c", subcore_axis_name="s")

    @functools.partial(
        pl.kernel, mesh=mesh,
        compiler_params=pltpu.CompilerParams(use_tc_tiling_on_sc=False),
        out_type=[jax.ShapeDtypeStruct((n, h), jnp.float32),
                  jax.ShapeDtypeStruct((n, h), jnp.float32)],
        scratch_types=[
            pltpu.VMEM((_CH,), jnp.int32),
            pltpu.VMEM((_CH, hh), jnp.float32),
            pltpu.VMEM((zb, hh), jnp.float32),
            pltpu.VMEM((512, hh), jnp.float32),
            pltpu.VMEM_SHARED((acc_rows, hh), jnp.float32),
            pltpu.SemaphoreType.DMA,
        ])
    def k(nef_hbm, ridx_hbm, seg_hbm, cnt_hbm, idxb, valb, zerob, oneb, acc,
          sem):
        core = lax.axis_index("c")
        tid = lax.axis_index("s")
        colbase = core * hh

        @pl.loop(0, zb)
        def _(i):
            zerob[i, pl.ds(0, hh)] = jnp.zeros((hh,), jnp.float32)

        @pl.loop(0, 512)
        def _(i):
            oneb[i, pl.ds(0, hh)] = jnp.ones((hh,), jnp.float32)

        def zero_acc():
            for z in range(stripe // zb):
                pltpu.sync_copy(zerob, acc.at[pl.ds(tid * stripe + z * zb, zb)])

        def flush(dst_hbm):
            @pl.when(tid < _NS - 1)
            def _():
                pltpu.sync_copy(
                    acc.at[pl.ds(tid * stripe, stripe)],
                    dst_hbm.at[pl.ds(tid * stripe, stripe),
                               pl.ds(colbase, hh)])

            @pl.when(tid == _NS - 1)
            def _():
                pltpu.sync_copy(
                    acc.at[pl.ds((_NS - 1) * stripe, last_flush)],
                    dst_hbm.at[pl.ds((_NS - 1) * stripe, last_flush),
                               pl.ds(colbase, hh)])

        zero_acc()
        plsc.subcore_barrier()

        @pl.loop(0, n_chunks)
        def _(ci):
            base = tid * per_t + ci * _CH
            base4 = base // 4
            pltpu.sync_copy(ridx_hbm.at[pl.ds(base, _CH)], idxb)
            cps = []
            for j in range(4):
                cps.append(pltpu.async_copy(
                    nef_hbm.at[pl.ds(base4, _CH // 4),
                               pl.ds(j * h + colbase, hh)],
                    valb.at[pl.ds(j * (_CH // 4), _CH // 4)], sem))
            for cp in cps:
                cp.wait()
            pltpu.sync_copy(valb, acc.at[idxb], add=True)

        plsc.subcore_barrier()
        flush(seg_hbm)
        plsc.subcore_barrier()
        zero_acc()
        plsc.subcore_barrier()

        @pl.loop(0, cnt_chunks)
        def _(ci):
            base = core * (e_pad // _NC) + tid * per_t_cnt + ci * _CH
            pltpu.sync_copy(ridx_hbm.at[pl.ds(base, _CH)], idxb)
            pltpu.sync_copy(oneb, acc.at[idxb.at[pl.ds(0, 512)]], add=True)
            pltpu.sync_copy(oneb, acc.at[idxb.at[pl.ds(512, 512)]], add=True)

        plsc.subcore_barrier()
        flush(cnt_hbm)

    return k(nef_p, r1d_perm)


def _ln_packed(x, mavg, g, b):
    m = jnp.dot(x, mavg, preferred_element_type=jnp.float32)
    d = x - m
    v = jnp.dot(d * d, mavg, preferred_element_type=jnp.float32)
    return d * jax.lax.rsqrt(v + 1e-5) * g + b


def _edge_body(x_ref, g_ref, w1bd, eeb1, w2bd, eeb2, eeg, eebe,
               w1cbd, pew2bd, peb2, peg, pebe, mavg_ref, nef_ref):
    x = x_ref[...]
    mavg = mavg_ref[...]
    h1 = _silu(jnp.dot(x, w1bd[...], preferred_element_type=jnp.float32)
               + eeb1[...])
    e1 = jnp.dot(h1, w2bd[...], preferred_element_type=jnp.float32) + eeb2[...]
    ef = _ln_packed(e1, mavg, eeg[...], eebe[...])
    pre = g_ref[...] + jnp.dot(ef, w1cbd[...],
                               preferred_element_type=jnp.float32)
    h2 = _silu(pre)
    e2 = jnp.dot(h2, pew2bd[...], preferred_element_type=jnp.float32) + peb2[...]
    nef_ref[...] = _ln_packed(e2, mavg, peg[...], pebe[...])


def _edge_pass(x_p, g_p, w1bd, eeb1, w2bd, eeb2, eeg, eebe,
               w1cbd, pew2bd, peb2, peg, pebe, mavg, block):
    ep4, de4 = x_p.shape
    grid = (ep4 // block,)
    full = lambda s: pl.BlockSpec(s, lambda i: (0,) * len(s))
    rowx = pl.BlockSpec((block, de4), lambda i: (i, 0))
    rowp = pl.BlockSpec((block, 128), lambda i: (i, 0))
    return pl.pallas_call(
        _edge_body,
        grid=grid,
        in_specs=[rowx, rowp,
                  full((de4, 128)), full((1, 128)), full((128, 128)),
                  full((1, 128)), full((1, 128)), full((1, 128)),
                  full((128, 128)), full((128, 128)), full((1, 128)),
                  full((1, 128)), full((1, 128)), full((128, 128))],
        out_specs=rowp,
        out_shape=jax.ShapeDtypeStruct((g_p.shape[0], 128), jnp.float32),
    )(x_p, g_p, w1bd, eeb1, w2bd, eeb2, eeg, eebe,
      w1cbd, pew2bd, peb2, peg, pebe, mavg)


def _node_body(nf_ref, seg_ref, cnt_ref, pnW1a, pnW1b, pnb1, pnW2, pnb2,
               png, pnbe, noW1, nob1, noW2, nob2, out_ref):
    cnt = cnt_ref[:, 0:1] + cnt_ref[:, 16:17]
    mean = seg_ref[...] / jnp.maximum(cnt, 1.0)
    nf = nf_ref[...]
    pre = (jnp.dot(nf, pnW1a[...], preferred_element_type=jnp.float32)
           + jnp.dot(mean, pnW1b[...], preferred_element_type=jnp.float32)
           + pnb1[...])
    hdd = _silu(pre)
    y = _ln(jnp.dot(hdd, pnW2[...], preferred_element_type=jnp.float32)
            + pnb2[...], png[...], pnbe[...])
    z = jax.nn.sigmoid(jnp.dot(y, noW1[...], preferred_element_type=jnp.float32)
                       + nob1[...])
    out_ref[...] = (jnp.dot(z, noW2[...], preferred_element_type=jnp.float32)
                    + nob2[...])


def _node_pass(node_features, seg_sum, counts, pn_W1, pn_b1, pn_W2, pn_b2,
               pn_g, pn_be, no_W1, no_b1, no_W2, no_b2, block):
    n, h = node_features.shape
    dout = no_W2.shape[1]
    grid = (n // block,)
    full = lambda s: pl.BlockSpec(s, lambda i: (0,) * len(s))
    rowh = pl.BlockSpec((block, h), lambda i: (i, 0))
    rowo = pl.BlockSpec((block, dout), lambda i: (i, 0))
    w1a, w1b = pn_W1[:h], pn_W1[h:]
    return pl.pallas_call(
        _node_body,
        grid=grid,
        in_specs=[rowh, rowh, rowh,
                  full((h, h)), full((h, h)), full((1, h)), full((h, h)),
                  full((1, h)), full((1, h)), full((1, h)),
                  full((h, h)), full((1, h)), full((h, dout)), full((1, dout))],
        out_specs=rowo,
        out_shape=jax.ShapeDtypeStruct((n, dout), jnp.float32),
    )(node_features, seg_sum, counts, w1a, w1b, pn_b1, pn_W2, pn_b2,
      pn_g, pn_be, no_W1, no_b1, no_W2, no_b2)


def kernel(edge_idx, edge_features, node_latents, node_features,
           ee_W1, ee_b1, ee_W2, ee_b2, ee_g, ee_be,
           pe_W1, pe_b1, pe_W2, pe_b2, pe_g, pe_be,
           pn_W1, pn_b1, pn_W2, pn_b2, pn_g, pn_be,
           no_W1, no_b1, no_W2, no_b2):
    e = edge_idx.shape[0]
    n, h = node_features.shape
    r2 = lambda v: v.reshape(1, -1)

    senders = edge_idx[:, 0]
    receivers = edge_idx[:, 1]
    e_pad = ((e + _CH * _NW - 1) // (_CH * _NW)) * (_CH * _NW)
    pad0 = jnp.zeros((e_pad - e,), jnp.int32)
    s1d = jnp.concatenate([senders, pad0])
    r1d_g = jnp.concatenate([receivers, pad0])
    r1d = jnp.concatenate([receivers, jnp.full((e_pad - e,), n, jnp.int32)])
    r1d_perm = r1d.reshape(-1, _CH // 4, 4).transpose(0, 2, 1).reshape(-1)

    from jax.scipy.linalg import block_diag
    bd4 = lambda w: block_diag(w, w, w, w)
    t4 = lambda v: jnp.tile(v, 4).reshape(1, -1)
    x_p = edge_features.reshape(e // 4, -1)
    mavg = bd4(jnp.full((h, h), 1.0 / h, jnp.float32))

    w1a, w1b, w1c = pe_W1[:h], pe_W1[h:2 * h], pe_W1[2 * h:]
    nl2, nf2 = _transform_tables(node_latents, node_features, w1a, w1b,
                                 r2(pe_b1), block=4000 if n % 4000 == 0 else n)

    g_p = _sc_gather_add(nl2, nf2, s1d, r1d_g)

    cvec = lambda v: v.reshape(-1, 1)
    eft = _ee_transposed(edge_features.T, ee_W1.T, cvec(ee_b1), ee_W2.T,
                         cvec(ee_b2), cvec(ee_g), cvec(ee_be),
                         block=6400 if e % 6400 == 0 else e)
    ef = eft.T

    nef_p = _edge_pass(x_p, g_p, bd4(ee_W1), t4(ee_b1), bd4(ee_W2),
                       t4(ee_b2), t4(ee_g), t4(ee_be), bd4(w1c),
                       bd4(pe_W2), t4(pe_b2), t4(pe_g), t4(pe_be),
                       mavg, block=2000 if (e // 4) % 2000 == 0 else e // 4)

    seg_sum, counts = _sc_scatter(nef_p, r1d_perm, n, h)

    out = _node_pass(node_features, seg_sum, counts, pn_W1, r2(pn_b1), pn_W2,
                     r2(pn_b2), r2(pn_g), r2(pn_be), no_W1, r2(no_b1), no_W2,
                     r2(no_b2), block=4000 if n % 4000 == 0 else n)
    return (ef, out)
